# baseline (speedup 1.0000x reference)
import functools

import jax
import jax.numpy as jnp
from jax import lax
from jax.experimental import pallas as pl
from jax.experimental.pallas import tpu as pltpu
from jax.experimental.pallas import tpu_sc as plsc

VOCAB = 50257
DIM = 512
NUM_HEADS = 8
HEAD_DIM = 64
NUM_EXPERTS = 8
D_FF = 1024
SEQ = 2048

TOK_BLK = 256
Q_BLK = 512
V_BLK = 2048
GATHER_PER_STEP = 8


def _embed_gather_sc(table, ids):
    nw = 32
    b_per_w = SEQ // nw
    mesh = plsc.VectorSubcoreMesh(core_axis_name="c", subcore_axis_name="s")

    @functools.partial(
        pl.kernel, mesh=mesh,
        out_type=jax.ShapeDtypeStruct((SEQ, DIM), jnp.float32),
        scratch_types=[
            pltpu.VMEM((b_per_w,), jnp.int32),
            pltpu.VMEM((b_per_w, DIM), jnp.float32),
            pltpu.SemaphoreType.DMA,
        ],
    )
    def k(ids_hbm, table_hbm, out_hbm, idx_v, rows_v, sem):
        wid = lax.axis_index("s") * 2 + lax.axis_index("c")
        base = wid * b_per_w
        pltpu.sync_copy(ids_hbm.at[pl.ds(base, b_per_w)], idx_v)
        pltpu.async_copy(table_hbm.at[idx_v], rows_v, sem).wait()
        pltpu.sync_copy(rows_v, out_hbm.at[pl.ds(base, b_per_w)])

    return k(ids, table)


def _embed_tc_body(ids_ref, *refs):
    i = pl.program_id(0)
    out_ref = refs[GATHER_PER_STEP]
    rows = []
    iota8 = lax.broadcasted_iota(jnp.int32, (8, DIM), 0)
    for j in range(GATHER_PER_STEP):
        sub = lax.rem(ids_ref[i * GATHER_PER_STEP + j], 8)
        blk = refs[j][...]
        rows.append(jnp.sum(jnp.where(iota8 == sub, blk, 0.0),
                            axis=0, keepdims=True))
    out_ref[...] = jnp.concatenate(rows, axis=0)


def _embed_gather_tc(table, ids):
    def tbl_spec(j):
        return pl.BlockSpec(
            (8, DIM),
            lambda i, ids_ref, j=j: (ids_ref[i * GATHER_PER_STEP + j] // 8,
                                     0))

    grid_spec = pltpu.PrefetchScalarGridSpec(
        num_scalar_prefetch=1,
        grid=(SEQ // GATHER_PER_STEP,),
        in_specs=[tbl_spec(j) for j in range(GATHER_PER_STEP)],
        out_specs=pl.BlockSpec((GATHER_PER_STEP, DIM),
                               lambda i, ids_ref: (i, 0)),
    )
    return pl.pallas_call(
        _embed_tc_body,
        grid_spec=grid_spec,
        out_shape=jax.ShapeDtypeStruct((SEQ, DIM), jnp.float32),
    )(ids, *([table] * GATHER_PER_STEP))


def _pre_body(x_ref, pos_ref, ow1_ref, ob1_ref, ow2_ref, ob2_ref,
              dw1_ref, db1_ref, dw2_ref, db2_ref, out_ref):
    x = x_ref[...] + pos_ref[...]
    ov = jax.nn.gelu(
        jnp.dot(x, ow1_ref[...], preferred_element_type=jnp.float32)
        + ob1_ref[...])
    ov = jnp.dot(ov, ow2_ref[...], preferred_element_type=jnp.float32) \
        + ob2_ref[...]
    x = x + 0.1 * ov
    xd = x
    for _ in range(5):
        h = jax.nn.gelu(
            jnp.dot(xd, dw1_ref[...], preferred_element_type=jnp.float32)
            + db1_ref[...])
        h = jnp.dot(h, dw2_ref[...], preferred_element_type=jnp.float32) \
            + db2_ref[...]
        xd = xd - 0.1 * h
    out_ref[...] = x + 0.2 * xd


def _pre_mlps(x, pos, p):
    full = lambda r, c: pl.BlockSpec((r, c), lambda i: (0, 0))
    return pl.pallas_call(
        _pre_body,
        grid=(SEQ // TOK_BLK,),
        in_specs=[
            pl.BlockSpec((TOK_BLK, DIM), lambda i: (i, 0)),
            pl.BlockSpec((TOK_BLK, DIM), lambda i: (i, 0)),
            full(DIM, 4 * DIM), full(1, 4 * DIM),
            full(4 * DIM, DIM), full(1, DIM),
            full(DIM, 2 * DIM), full(1, 2 * DIM),
            full(2 * DIM, DIM), full(1, DIM),
        ],
        out_specs=pl.BlockSpec((TOK_BLK, DIM), lambda i: (i, 0)),
        out_shape=jax.ShapeDtypeStruct((SEQ, DIM), jnp.float32),
    )(x, pos, p['ov_w1'], p['ov_b1'][None, :], p['ov_w2'],
      p['ov_b2'][None, :], p['df_w1'], p['df_b1'][None, :], p['df_w2'],
      p['df_b2'][None, :])


def _matmul_bias_body(x_ref, w_ref, b_ref, out_ref):
    out_ref[...] = jnp.dot(x_ref[...], w_ref[...],
                           preferred_element_type=jnp.float32) + b_ref[...]


def _matmul_bias(x, w, b):
    n = w.shape[1]
    return pl.pallas_call(
        _matmul_bias_body,
        grid=(SEQ // TOK_BLK,),
        in_specs=[
            pl.BlockSpec((TOK_BLK, DIM), lambda i: (i, 0)),
            pl.BlockSpec((DIM, n), lambda i: (0, 0)),
            pl.BlockSpec((1, n), lambda i: (0, 0)),
        ],
        out_specs=pl.BlockSpec((TOK_BLK, n), lambda i: (i, 0)),
        out_shape=jax.ShapeDtypeStruct((SEQ, n), jnp.float32),
    )(x, w, b[None, :])


def _attn_body(q_ref, k_ref, v_ref, out_ref):
    qb = pl.program_id(0)
    q_all = q_ref[...]
    k_all = k_ref[...]
    v_all = v_ref[...]
    row = qb * Q_BLK + lax.broadcasted_iota(jnp.int32, (Q_BLK, SEQ), 0)
    col = lax.broadcasted_iota(jnp.int32, (Q_BLK, SEQ), 1)
    causal = row >= col
    outs = []
    for h in range(NUM_HEADS):
        q = q_all[:, h * HEAD_DIM:(h + 1) * HEAD_DIM]
        k = k_all[:, h * HEAD_DIM:(h + 1) * HEAD_DIM]
        v = v_all[:, h * HEAD_DIM:(h + 1) * HEAD_DIM]
        s = lax.dot_general(q, k, (((1,), (1,)), ((), ())),
                            preferred_element_type=jnp.float32) / 8.0
        s = jnp.where(causal, s, jnp.float32(-1e9))
        m = jnp.max(s, axis=-1, keepdims=True)
        e = jnp.exp(s - m)
        p = e / jnp.sum(e, axis=-1, keepdims=True)
        outs.append(jnp.dot(p, v, preferred_element_type=jnp.float32))
    out_ref[...] = jnp.concatenate(outs, axis=1)


def _attention(x, lp):
    qkv = _matmul_bias(x, lp['wqkv'], lp['bqkv'])
    heads = pl.pallas_call(
        _attn_body,
        grid=(SEQ // Q_BLK,),
        in_specs=[
            pl.BlockSpec((Q_BLK, DIM), lambda qb: (qb, 0)),
            pl.BlockSpec((SEQ, DIM), lambda qb: (0, 1)),
            pl.BlockSpec((SEQ, DIM), lambda qb: (0, 2)),
        ],
        out_specs=pl.BlockSpec((Q_BLK, DIM), lambda qb: (qb, 0)),
        out_shape=jax.ShapeDtypeStruct((SEQ, DIM), jnp.float32),
    )(qkv, qkv, qkv)
    return heads


def _proj_residual_body(h_ref, w_ref, b_ref, x_ref, out_ref):
    out_ref[...] = (jnp.dot(h_ref[...], w_ref[...],
                            preferred_element_type=jnp.float32)
                    + b_ref[...] + x_ref[...])


def _proj_residual(heads, w, b, x):
    return pl.pallas_call(
        _proj_residual_body,
        grid=(SEQ // TOK_BLK,),
        in_specs=[
            pl.BlockSpec((TOK_BLK, DIM), lambda i: (i, 0)),
            pl.BlockSpec((DIM, DIM), lambda i: (0, 0)),
            pl.BlockSpec((1, DIM), lambda i: (0, 0)),
            pl.BlockSpec((TOK_BLK, DIM), lambda i: (i, 0)),
        ],
        out_specs=pl.BlockSpec((TOK_BLK, DIM), lambda i: (i, 0)),
        out_shape=jax.ShapeDtypeStruct((SEQ, DIM), jnp.float32),
    )(heads, w, b[None, :], x)


def _moe_body(x_ref, wr_ref, w1_ref, b1_ref, w2_ref, b2_ref, out_ref):
    e = pl.program_id(0)
    x = x_ref[...]
    logits = jnp.dot(x, wr_ref[...], preferred_element_type=jnp.float32)
    lm = jnp.max(logits, axis=-1, keepdims=True)
    ex = jnp.exp(logits - lm)
    probs = ex / jnp.sum(ex, axis=-1, keepdims=True)
    iota = lax.broadcasted_iota(jnp.int32, probs.shape, 1)
    m1 = jnp.max(probs, axis=-1, keepdims=True)
    i1 = jnp.min(jnp.where(probs == m1, iota, NUM_EXPERTS),
                 axis=-1, keepdims=True)
    pm = jnp.where(iota == i1, jnp.float32(-1.0), probs)
    m2 = jnp.max(pm, axis=-1, keepdims=True)
    i2 = jnp.min(jnp.where(pm == m2, iota, NUM_EXPERTS),
                 axis=-1, keepdims=True)
    denom = m1 + m2
    w_e = jnp.where(i1 == e, m1 / denom,
                    jnp.where(i2 == e, m2 / denom, jnp.float32(0.0)))

    h = jax.nn.gelu(
        jnp.dot(x, w1_ref[0], preferred_element_type=jnp.float32)
        + b1_ref[0])
    y = jnp.dot(h, w2_ref[0], preferred_element_type=jnp.float32) \
        + b2_ref[0]

    @pl.when(e == 0)
    def _():
        out_ref[...] = x

    out_ref[...] += w_e * y


def _mini_moe(x, mp):
    return pl.pallas_call(
        _moe_body,
        grid=(NUM_EXPERTS,),
        in_specs=[
            pl.BlockSpec((SEQ, DIM), lambda e: (0, 0)),
            pl.BlockSpec((DIM, NUM_EXPERTS), lambda e: (0, 0)),
            pl.BlockSpec((1, DIM, D_FF), lambda e: (e, 0, 0)),
            pl.BlockSpec((1, 1, D_FF), lambda e: (e, 0, 0)),
            pl.BlockSpec((1, D_FF, DIM), lambda e: (e, 0, 0)),
            pl.BlockSpec((1, 1, DIM), lambda e: (e, 0, 0)),
        ],
        out_specs=pl.BlockSpec((SEQ, DIM), lambda e: (0, 0)),
        out_shape=jax.ShapeDtypeStruct((SEQ, DIM), jnp.float32),
    )(x, mp['wr'], mp['w1'], mp['b1'][:, None, :], mp['w2'],
      mp['b2'][:, None, :])


def _final_body(x_ref, g_ref, b_ref, tbl_ref, out_ref):
    x = x_ref[...]
    mu = jnp.mean(x, axis=-1, keepdims=True)
    var = jnp.mean((x - mu) ** 2, axis=-1, keepdims=True)
    xn = (x - mu) / jnp.sqrt(var + 1e-5) * g_ref[...] + b_ref[...]
    out_ref[...] = lax.dot_general(xn, tbl_ref[...],
                                   (((1,), (1,)), ((), ())),
                                   preferred_element_type=jnp.float32)


def _final_logits(x, g, b, table):
    return pl.pallas_call(
        _final_body,
        grid=(pl.cdiv(VOCAB, V_BLK),),
        in_specs=[
            pl.BlockSpec((SEQ, DIM), lambda j: (0, 0)),
            pl.BlockSpec((1, DIM), lambda j: (0, 0)),
            pl.BlockSpec((1, DIM), lambda j: (0, 0)),
            pl.BlockSpec((V_BLK, DIM), lambda j: (j, 0)),
        ],
        out_specs=pl.BlockSpec((SEQ, V_BLK), lambda j: (0, j)),
        out_shape=jax.ShapeDtypeStruct((SEQ, VOCAB), jnp.float32),
    )(x, g[None, :], b[None, :], table)


def kernel(params, input_ids):
    ids = input_ids.reshape(SEQ).astype(jnp.int32)
    pos = params['pos_embed'][0, :SEQ, :]
    x = _embed_gather_tc(params['token_embed'], ids)
    x = _pre_mlps(x, pos, params)
    for lp in params['layers']:
        heads = _attention(x, lp)
        x = _proj_residual(heads, lp['wo'], lp['bo'], x)
        for mp in lp['moes']:
            x = _mini_moe(x, mp)
    logits = _final_logits(x, params['ln_g'], params['ln_b'],
                           params['token_embed'])
    return logits.reshape(1, SEQ, VOCAB)


# speedup vs baseline: 1.0676x; 1.0676x over previous
import functools

import jax
import jax.numpy as jnp
from jax import lax
from jax.experimental import pallas as pl
from jax.experimental.pallas import tpu as pltpu
from jax.experimental.pallas import tpu_sc as plsc

VOCAB = 50257
DIM = 512
NUM_HEADS = 8
HEAD_DIM = 64
NUM_EXPERTS = 8
D_FF = 1024
SEQ = 2048

TOK_BLK = 256
Q_BLK = 512
V_BLK = 2048
GATHER_PER_STEP = 32


def _embed_gather_sc(table, ids):
    nw = 32
    b_per_w = SEQ // nw
    mesh = plsc.VectorSubcoreMesh(core_axis_name="c", subcore_axis_name="s")

    @functools.partial(
        pl.kernel, mesh=mesh,
        out_type=jax.ShapeDtypeStruct((SEQ, DIM), jnp.float32),
        scratch_types=[
            pltpu.VMEM((b_per_w,), jnp.int32),
            pltpu.VMEM((b_per_w, DIM), jnp.float32),
            pltpu.SemaphoreType.DMA,
        ],
    )
    def k(ids_hbm, table_hbm, out_hbm, idx_v, rows_v, sem):
        wid = lax.axis_index("s") * 2 + lax.axis_index("c")
        base = wid * b_per_w
        pltpu.sync_copy(ids_hbm.at[pl.ds(base, b_per_w)], idx_v)
        pltpu.async_copy(table_hbm.at[idx_v], rows_v, sem).wait()
        pltpu.sync_copy(rows_v, out_hbm.at[pl.ds(base, b_per_w)])

    return k(ids, table)


def _embed_tc_body(ids_ref, *refs):
    i = pl.program_id(0)
    out_ref = refs[GATHER_PER_STEP]
    rows = []
    iota8 = lax.broadcasted_iota(jnp.int32, (8, DIM), 0)
    for j in range(GATHER_PER_STEP):
        sub = lax.rem(ids_ref[i * GATHER_PER_STEP + j], 8)
        blk = refs[j][...]
        rows.append(jnp.sum(jnp.where(iota8 == sub, blk, 0.0),
                            axis=0, keepdims=True))
    out_ref[...] = jnp.concatenate(rows, axis=0)


def _embed_gather_tc(table, ids):
    def tbl_spec(j):
        return pl.BlockSpec(
            (8, DIM),
            lambda i, ids_ref, j=j: (ids_ref[i * GATHER_PER_STEP + j] // 8,
                                     0))

    grid_spec = pltpu.PrefetchScalarGridSpec(
        num_scalar_prefetch=1,
        grid=(SEQ // GATHER_PER_STEP,),
        in_specs=[tbl_spec(j) for j in range(GATHER_PER_STEP)],
        out_specs=pl.BlockSpec((GATHER_PER_STEP, DIM),
                               lambda i, ids_ref: (i, 0)),
    )
    return pl.pallas_call(
        _embed_tc_body,
        grid_spec=grid_spec,
        out_shape=jax.ShapeDtypeStruct((SEQ, DIM), jnp.float32),
    )(ids, *([table] * GATHER_PER_STEP))


def _pre_body(x_ref, pos_ref, ow1_ref, ob1_ref, ow2_ref, ob2_ref,
              dw1_ref, db1_ref, dw2_ref, db2_ref, out_ref):
    x = x_ref[...] + pos_ref[...]
    ov = jax.nn.gelu(
        jnp.dot(x, ow1_ref[...], preferred_element_type=jnp.float32)
        + ob1_ref[...])
    ov = jnp.dot(ov, ow2_ref[...], preferred_element_type=jnp.float32) \
        + ob2_ref[...]
    x = x + 0.1 * ov
    xd = x
    for _ in range(5):
        h = jax.nn.gelu(
            jnp.dot(xd, dw1_ref[...], preferred_element_type=jnp.float32)
            + db1_ref[...])
        h = jnp.dot(h, dw2_ref[...], preferred_element_type=jnp.float32) \
            + db2_ref[...]
        xd = xd - 0.1 * h
    out_ref[...] = x + 0.2 * xd


def _pre_mlps(x, pos, p):
    full = lambda r, c: pl.BlockSpec((r, c), lambda i: (0, 0))
    return pl.pallas_call(
        _pre_body,
        grid=(SEQ // TOK_BLK,),
        in_specs=[
            pl.BlockSpec((TOK_BLK, DIM), lambda i: (i, 0)),
            pl.BlockSpec((TOK_BLK, DIM), lambda i: (i, 0)),
            full(DIM, 4 * DIM), full(1, 4 * DIM),
            full(4 * DIM, DIM), full(1, DIM),
            full(DIM, 2 * DIM), full(1, 2 * DIM),
            full(2 * DIM, DIM), full(1, DIM),
        ],
        out_specs=pl.BlockSpec((TOK_BLK, DIM), lambda i: (i, 0)),
        out_shape=jax.ShapeDtypeStruct((SEQ, DIM), jnp.float32),
    )(x, pos, p['ov_w1'], p['ov_b1'][None, :], p['ov_w2'],
      p['ov_b2'][None, :], p['df_w1'], p['df_b1'][None, :], p['df_w2'],
      p['df_b2'][None, :])


def _matmul_bias_body(x_ref, w_ref, b_ref, out_ref):
    out_ref[...] = jnp.dot(x_ref[...], w_ref[...],
                           preferred_element_type=jnp.float32) + b_ref[...]


def _matmul_bias(x, w, b):
    n = w.shape[1]
    return pl.pallas_call(
        _matmul_bias_body,
        grid=(SEQ // TOK_BLK,),
        in_specs=[
            pl.BlockSpec((TOK_BLK, DIM), lambda i: (i, 0)),
            pl.BlockSpec((DIM, n), lambda i: (0, 0)),
            pl.BlockSpec((1, n), lambda i: (0, 0)),
        ],
        out_specs=pl.BlockSpec((TOK_BLK, n), lambda i: (i, 0)),
        out_shape=jax.ShapeDtypeStruct((SEQ, n), jnp.float32),
    )(x, w, b[None, :])


def _attn_body(q_ref, k_ref, v_ref, out_ref):
    qb = pl.program_id(0)
    q_all = q_ref[...]
    k_all = k_ref[...]
    v_all = v_ref[...]
    row = qb * Q_BLK + lax.broadcasted_iota(jnp.int32, (Q_BLK, SEQ), 0)
    col = lax.broadcasted_iota(jnp.int32, (Q_BLK, SEQ), 1)
    causal = row >= col
    outs = []
    for h in range(NUM_HEADS):
        q = q_all[:, h * HEAD_DIM:(h + 1) * HEAD_DIM]
        k = k_all[:, h * HEAD_DIM:(h + 1) * HEAD_DIM]
        v = v_all[:, h * HEAD_DIM:(h + 1) * HEAD_DIM]
        s = lax.dot_general(q, k, (((1,), (1,)), ((), ())),
                            preferred_element_type=jnp.float32) / 8.0
        s = jnp.where(causal, s, jnp.float32(-1e9))
        m = jnp.max(s, axis=-1, keepdims=True)
        e = jnp.exp(s - m)
        p = e / jnp.sum(e, axis=-1, keepdims=True)
        outs.append(jnp.dot(p, v, preferred_element_type=jnp.float32))
    out_ref[...] = jnp.concatenate(outs, axis=1)


def _attention(x, lp):
    qkv = _matmul_bias(x, lp['wqkv'], lp['bqkv'])
    heads = pl.pallas_call(
        _attn_body,
        grid=(SEQ // Q_BLK,),
        in_specs=[
            pl.BlockSpec((Q_BLK, DIM), lambda qb: (qb, 0)),
            pl.BlockSpec((SEQ, DIM), lambda qb: (0, 1)),
            pl.BlockSpec((SEQ, DIM), lambda qb: (0, 2)),
        ],
        out_specs=pl.BlockSpec((Q_BLK, DIM), lambda qb: (qb, 0)),
        out_shape=jax.ShapeDtypeStruct((SEQ, DIM), jnp.float32),
    )(qkv, qkv, qkv)
    return heads


def _proj_residual_body(h_ref, w_ref, b_ref, x_ref, out_ref):
    out_ref[...] = (jnp.dot(h_ref[...], w_ref[...],
                            preferred_element_type=jnp.float32)
                    + b_ref[...] + x_ref[...])


def _proj_residual(heads, w, b, x):
    return pl.pallas_call(
        _proj_residual_body,
        grid=(SEQ // TOK_BLK,),
        in_specs=[
            pl.BlockSpec((TOK_BLK, DIM), lambda i: (i, 0)),
            pl.BlockSpec((DIM, DIM), lambda i: (0, 0)),
            pl.BlockSpec((1, DIM), lambda i: (0, 0)),
            pl.BlockSpec((TOK_BLK, DIM), lambda i: (i, 0)),
        ],
        out_specs=pl.BlockSpec((TOK_BLK, DIM), lambda i: (i, 0)),
        out_shape=jax.ShapeDtypeStruct((SEQ, DIM), jnp.float32),
    )(heads, w, b[None, :], x)


def _moe_body(x_ref, wr_ref, w1_ref, b1_ref, w2_ref, b2_ref, out_ref):
    e = pl.program_id(0)
    x = x_ref[...]
    logits = jnp.dot(x, wr_ref[...], preferred_element_type=jnp.float32)
    lm = jnp.max(logits, axis=-1, keepdims=True)
    ex = jnp.exp(logits - lm)
    probs = ex / jnp.sum(ex, axis=-1, keepdims=True)
    iota = lax.broadcasted_iota(jnp.int32, probs.shape, 1)
    m1 = jnp.max(probs, axis=-1, keepdims=True)
    i1 = jnp.min(jnp.where(probs == m1, iota, NUM_EXPERTS),
                 axis=-1, keepdims=True)
    pm = jnp.where(iota == i1, jnp.float32(-1.0), probs)
    m2 = jnp.max(pm, axis=-1, keepdims=True)
    i2 = jnp.min(jnp.where(pm == m2, iota, NUM_EXPERTS),
                 axis=-1, keepdims=True)
    denom = m1 + m2
    w_e = jnp.where(i1 == e, m1 / denom,
                    jnp.where(i2 == e, m2 / denom, jnp.float32(0.0)))

    h = jax.nn.gelu(
        jnp.dot(x, w1_ref[0], preferred_element_type=jnp.float32)
        + b1_ref[0])
    y = jnp.dot(h, w2_ref[0], preferred_element_type=jnp.float32) \
        + b2_ref[0]

    @pl.when(e == 0)
    def _():
        out_ref[...] = x

    out_ref[...] += w_e * y


def _mini_moe(x, mp):
    return pl.pallas_call(
        _moe_body,
        grid=(NUM_EXPERTS,),
        in_specs=[
            pl.BlockSpec((SEQ, DIM), lambda e: (0, 0)),
            pl.BlockSpec((DIM, NUM_EXPERTS), lambda e: (0, 0)),
            pl.BlockSpec((1, DIM, D_FF), lambda e: (e, 0, 0)),
            pl.BlockSpec((1, 1, D_FF), lambda e: (e, 0, 0)),
            pl.BlockSpec((1, D_FF, DIM), lambda e: (e, 0, 0)),
            pl.BlockSpec((1, 1, DIM), lambda e: (e, 0, 0)),
        ],
        out_specs=pl.BlockSpec((SEQ, DIM), lambda e: (0, 0)),
        out_shape=jax.ShapeDtypeStruct((SEQ, DIM), jnp.float32),
    )(x, mp['wr'], mp['w1'], mp['b1'][:, None, :], mp['w2'],
      mp['b2'][:, None, :])


def _final_body(x_ref, g_ref, b_ref, tbl_ref, out_ref):
    x = x_ref[...]
    mu = jnp.mean(x, axis=-1, keepdims=True)
    var = jnp.mean((x - mu) ** 2, axis=-1, keepdims=True)
    xn = (x - mu) / jnp.sqrt(var + 1e-5) * g_ref[...] + b_ref[...]
    out_ref[...] = lax.dot_general(xn, tbl_ref[...],
                                   (((1,), (1,)), ((), ())),
                                   preferred_element_type=jnp.float32)


def _final_logits(x, g, b, table):
    return pl.pallas_call(
        _final_body,
        grid=(pl.cdiv(VOCAB, V_BLK),),
        in_specs=[
            pl.BlockSpec((SEQ, DIM), lambda j: (0, 0)),
            pl.BlockSpec((1, DIM), lambda j: (0, 0)),
            pl.BlockSpec((1, DIM), lambda j: (0, 0)),
            pl.BlockSpec((V_BLK, DIM), lambda j: (j, 0)),
        ],
        out_specs=pl.BlockSpec((SEQ, V_BLK), lambda j: (0, j)),
        out_shape=jax.ShapeDtypeStruct((SEQ, VOCAB), jnp.float32),
    )(x, g[None, :], b[None, :], table)


def kernel(params, input_ids):
    ids = input_ids.reshape(SEQ).astype(jnp.int32)
    pos = params['pos_embed'][0, :SEQ, :]
    x = _embed_gather_tc(params['token_embed'], ids)
    x = _pre_mlps(x, pos, params)
    for lp in params['layers']:
        heads = _attention(x, lp)
        x = _proj_residual(heads, lp['wo'], lp['bo'], x)
        for mp in lp['moes']:
            x = _mini_moe(x, mp)
    logits = _final_logits(x, params['ln_g'], params['ln_b'],
                           params['token_embed'])
    return logits.reshape(1, SEQ, VOCAB)


# speedup vs baseline: 1.1461x; 1.0736x over previous
import functools

import jax
import jax.numpy as jnp
from jax import lax
from jax.experimental import pallas as pl
from jax.experimental.pallas import tpu as pltpu
from jax.experimental.pallas import tpu_sc as plsc

VOCAB = 50257
DIM = 512
NUM_HEADS = 8
HEAD_DIM = 64
NUM_EXPERTS = 8
D_FF = 1024
SEQ = 2048

TOK_BLK = 256
Q_BLK = 512
V_BLK = 2048
GATHER_PER_STEP = 32


def _embed_gather_sc(table, ids):
    nw = 32
    b_per_w = SEQ // nw
    mesh = plsc.VectorSubcoreMesh(core_axis_name="c", subcore_axis_name="s")

    @functools.partial(
        pl.kernel, mesh=mesh,
        out_type=jax.ShapeDtypeStruct((SEQ, DIM), jnp.float32),
        scratch_types=[
            pltpu.VMEM((b_per_w,), jnp.int32),
            pltpu.VMEM((b_per_w, DIM), jnp.float32),
            pltpu.SemaphoreType.DMA,
        ],
    )
    def k(ids_hbm, table_hbm, out_hbm, idx_v, rows_v, sem):
        wid = lax.axis_index("s") * 2 + lax.axis_index("c")
        base = wid * b_per_w
        pltpu.sync_copy(ids_hbm.at[pl.ds(base, b_per_w)], idx_v)
        pltpu.async_copy(table_hbm.at[idx_v], rows_v, sem).wait()
        pltpu.sync_copy(rows_v, out_hbm.at[pl.ds(base, b_per_w)])

    return k(ids, table)


def _embed_tc_body(ids_ref, *refs):
    i = pl.program_id(0)
    out_ref = refs[GATHER_PER_STEP]
    rows = []
    iota8 = lax.broadcasted_iota(jnp.int32, (8, DIM), 0)
    for j in range(GATHER_PER_STEP):
        sub = lax.rem(ids_ref[i * GATHER_PER_STEP + j], 8)
        blk = refs[j][...]
        rows.append(jnp.sum(jnp.where(iota8 == sub, blk, 0.0),
                            axis=0, keepdims=True))
    out_ref[...] = jnp.concatenate(rows, axis=0)


def _embed_gather_tc(table, ids):
    def tbl_spec(j):
        return pl.BlockSpec(
            (8, DIM),
            lambda i, ids_ref, j=j: (ids_ref[i * GATHER_PER_STEP + j] // 8,
                                     0))

    grid_spec = pltpu.PrefetchScalarGridSpec(
        num_scalar_prefetch=1,
        grid=(SEQ // GATHER_PER_STEP,),
        in_specs=[tbl_spec(j) for j in range(GATHER_PER_STEP)],
        out_specs=pl.BlockSpec((GATHER_PER_STEP, DIM),
                               lambda i, ids_ref: (i, 0)),
    )
    return pl.pallas_call(
        _embed_tc_body,
        grid_spec=grid_spec,
        out_shape=jax.ShapeDtypeStruct((SEQ, DIM), jnp.float32),
    )(ids, *([table] * GATHER_PER_STEP))


def _pre_body(x_ref, pos_ref, ow1_ref, ob1_ref, ow2_ref, ob2_ref,
              dw1_ref, db1_ref, dw2_ref, db2_ref, out_ref):
    x = x_ref[...] + pos_ref[...]
    ov = jax.nn.gelu(
        jnp.dot(x, ow1_ref[...], preferred_element_type=jnp.float32)
        + ob1_ref[...])
    ov = jnp.dot(ov, ow2_ref[...], preferred_element_type=jnp.float32) \
        + ob2_ref[...]
    x = x + 0.1 * ov
    xd = x
    for _ in range(5):
        h = jax.nn.gelu(
            jnp.dot(xd, dw1_ref[...], preferred_element_type=jnp.float32)
            + db1_ref[...])
        h = jnp.dot(h, dw2_ref[...], preferred_element_type=jnp.float32) \
            + db2_ref[...]
        xd = xd - 0.1 * h
    out_ref[...] = x + 0.2 * xd


def _pre_mlps(x, pos, p):
    full = lambda r, c: pl.BlockSpec((r, c), lambda i: (0, 0))
    return pl.pallas_call(
        _pre_body,
        grid=(SEQ // TOK_BLK,),
        in_specs=[
            pl.BlockSpec((TOK_BLK, DIM), lambda i: (i, 0)),
            pl.BlockSpec((TOK_BLK, DIM), lambda i: (i, 0)),
            full(DIM, 4 * DIM), full(1, 4 * DIM),
            full(4 * DIM, DIM), full(1, DIM),
            full(DIM, 2 * DIM), full(1, 2 * DIM),
            full(2 * DIM, DIM), full(1, DIM),
        ],
        out_specs=pl.BlockSpec((TOK_BLK, DIM), lambda i: (i, 0)),
        out_shape=jax.ShapeDtypeStruct((SEQ, DIM), jnp.float32),
    )(x, pos, p['ov_w1'], p['ov_b1'][None, :], p['ov_w2'],
      p['ov_b2'][None, :], p['df_w1'], p['df_b1'][None, :], p['df_w2'],
      p['df_b2'][None, :])


def _matmul_bias_body(x_ref, w_ref, b_ref, out_ref):
    out_ref[...] = jnp.dot(x_ref[...], w_ref[...],
                           preferred_element_type=jnp.float32) + b_ref[...]


def _matmul_bias(x, w, b):
    n = w.shape[1]
    return pl.pallas_call(
        _matmul_bias_body,
        grid=(SEQ // TOK_BLK,),
        in_specs=[
            pl.BlockSpec((TOK_BLK, DIM), lambda i: (i, 0)),
            pl.BlockSpec((DIM, n), lambda i: (0, 0)),
            pl.BlockSpec((1, n), lambda i: (0, 0)),
        ],
        out_specs=pl.BlockSpec((TOK_BLK, n), lambda i: (i, 0)),
        out_shape=jax.ShapeDtypeStruct((SEQ, n), jnp.float32),
    )(x, w, b[None, :])


def _attn_body(q_ref, k_ref, v_ref, out_ref):
    qb = pl.program_id(0)
    q_all = q_ref[...]
    k_all = k_ref[...]
    v_all = v_ref[...]
    row = qb * Q_BLK + lax.broadcasted_iota(jnp.int32, (Q_BLK, SEQ), 0)
    col = lax.broadcasted_iota(jnp.int32, (Q_BLK, SEQ), 1)
    causal = row >= col
    outs = []
    for h in range(NUM_HEADS):
        q = q_all[:, h * HEAD_DIM:(h + 1) * HEAD_DIM]
        k = k_all[:, h * HEAD_DIM:(h + 1) * HEAD_DIM]
        v = v_all[:, h * HEAD_DIM:(h + 1) * HEAD_DIM]
        s = lax.dot_general(q, k, (((1,), (1,)), ((), ())),
                            preferred_element_type=jnp.float32) / 8.0
        s = jnp.where(causal, s, jnp.float32(-1e9))
        m = jnp.max(s, axis=-1, keepdims=True)
        e = jnp.exp(s - m)
        p = e / jnp.sum(e, axis=-1, keepdims=True)
        outs.append(jnp.dot(p, v, preferred_element_type=jnp.float32))
    out_ref[...] = jnp.concatenate(outs, axis=1)


def _attention(x, lp):
    qkv = _matmul_bias(x, lp['wqkv'], lp['bqkv'])
    heads = pl.pallas_call(
        _attn_body,
        grid=(SEQ // Q_BLK,),
        in_specs=[
            pl.BlockSpec((Q_BLK, DIM), lambda qb: (qb, 0)),
            pl.BlockSpec((SEQ, DIM), lambda qb: (0, 1)),
            pl.BlockSpec((SEQ, DIM), lambda qb: (0, 2)),
        ],
        out_specs=pl.BlockSpec((Q_BLK, DIM), lambda qb: (qb, 0)),
        out_shape=jax.ShapeDtypeStruct((SEQ, DIM), jnp.float32),
    )(qkv, qkv, qkv)
    return heads


def _proj_residual_body(h_ref, w_ref, b_ref, x_ref, out_ref):
    out_ref[...] = (jnp.dot(h_ref[...], w_ref[...],
                            preferred_element_type=jnp.float32)
                    + b_ref[...] + x_ref[...])


def _proj_residual(heads, w, b, x):
    return pl.pallas_call(
        _proj_residual_body,
        grid=(SEQ // TOK_BLK,),
        in_specs=[
            pl.BlockSpec((TOK_BLK, DIM), lambda i: (i, 0)),
            pl.BlockSpec((DIM, DIM), lambda i: (0, 0)),
            pl.BlockSpec((1, DIM), lambda i: (0, 0)),
            pl.BlockSpec((TOK_BLK, DIM), lambda i: (i, 0)),
        ],
        out_specs=pl.BlockSpec((TOK_BLK, DIM), lambda i: (i, 0)),
        out_shape=jax.ShapeDtypeStruct((SEQ, DIM), jnp.float32),
    )(heads, w, b[None, :], x)


def _moe_body(x_ref, wr_ref, w1_ref, b1_ref, w2_ref, b2_ref, out_ref):
    e = pl.program_id(0)
    x = x_ref[...]
    logits = jnp.dot(x, wr_ref[...], preferred_element_type=jnp.float32)
    lm = jnp.max(logits, axis=-1, keepdims=True)
    ex = jnp.exp(logits - lm)
    probs = ex / jnp.sum(ex, axis=-1, keepdims=True)
    iota = lax.broadcasted_iota(jnp.int32, probs.shape, 1)
    m1 = jnp.max(probs, axis=-1, keepdims=True)
    i1 = jnp.min(jnp.where(probs == m1, iota, NUM_EXPERTS),
                 axis=-1, keepdims=True)
    pm = jnp.where(iota == i1, jnp.float32(-1.0), probs)
    m2 = jnp.max(pm, axis=-1, keepdims=True)
    i2 = jnp.min(jnp.where(pm == m2, iota, NUM_EXPERTS),
                 axis=-1, keepdims=True)
    denom = m1 + m2
    w_e = jnp.where(i1 == e, m1 / denom,
                    jnp.where(i2 == e, m2 / denom, jnp.float32(0.0)))

    xb = x.astype(jnp.bfloat16)
    w1b = w1_ref[0].astype(jnp.bfloat16)
    h = jax.nn.gelu(
        jnp.dot(xb, w1b, preferred_element_type=jnp.float32) + b1_ref[0])
    y = jnp.dot(h.astype(jnp.bfloat16), w2_ref[0].astype(jnp.bfloat16),
                preferred_element_type=jnp.float32) + b2_ref[0]

    @pl.when(e == 0)
    def _():
        out_ref[...] = x

    out_ref[...] += w_e * y


def _mini_moe(x, mp):
    return pl.pallas_call(
        _moe_body,
        grid=(NUM_EXPERTS,),
        in_specs=[
            pl.BlockSpec((SEQ, DIM), lambda e: (0, 0)),
            pl.BlockSpec((DIM, NUM_EXPERTS), lambda e: (0, 0)),
            pl.BlockSpec((1, DIM, D_FF), lambda e: (e, 0, 0)),
            pl.BlockSpec((1, 1, D_FF), lambda e: (e, 0, 0)),
            pl.BlockSpec((1, D_FF, DIM), lambda e: (e, 0, 0)),
            pl.BlockSpec((1, 1, DIM), lambda e: (e, 0, 0)),
        ],
        out_specs=pl.BlockSpec((SEQ, DIM), lambda e: (0, 0)),
        out_shape=jax.ShapeDtypeStruct((SEQ, DIM), jnp.float32),
    )(x, mp['wr'], mp['w1'], mp['b1'][:, None, :], mp['w2'],
      mp['b2'][:, None, :])


def _final_body(x_ref, g_ref, b_ref, tbl_ref, out_ref):
    x = x_ref[...]
    mu = jnp.mean(x, axis=-1, keepdims=True)
    var = jnp.mean((x - mu) ** 2, axis=-1, keepdims=True)
    xn = (x - mu) / jnp.sqrt(var + 1e-5) * g_ref[...] + b_ref[...]
    out_ref[...] = lax.dot_general(xn.astype(jnp.bfloat16),
                                   tbl_ref[...].astype(jnp.bfloat16),
                                   (((1,), (1,)), ((), ())),
                                   preferred_element_type=jnp.float32)


def _final_logits(x, g, b, table):
    return pl.pallas_call(
        _final_body,
        grid=(pl.cdiv(VOCAB, V_BLK),),
        in_specs=[
            pl.BlockSpec((SEQ, DIM), lambda j: (0, 0)),
            pl.BlockSpec((1, DIM), lambda j: (0, 0)),
            pl.BlockSpec((1, DIM), lambda j: (0, 0)),
            pl.BlockSpec((V_BLK, DIM), lambda j: (j, 0)),
        ],
        out_specs=pl.BlockSpec((SEQ, V_BLK), lambda j: (0, j)),
        out_shape=jax.ShapeDtypeStruct((SEQ, VOCAB), jnp.float32),
    )(x, g[None, :], b[None, :], table)


def kernel(params, input_ids):
    ids = input_ids.reshape(SEQ).astype(jnp.int32)
    pos = params['pos_embed'][0, :SEQ, :]
    x = _embed_gather_sc(params['token_embed'], ids)
    x = _pre_mlps(x, pos, params)
    for lp in params['layers']:
        heads = _attention(x, lp)
        x = _proj_residual(heads, lp['wo'], lp['bo'], x)
        for mp in lp['moes']:
            x = _mini_moe(x, mp)
    logits = _final_logits(x, params['ln_g'], params['ln_b'],
                           params['token_embed'])
    return logits.reshape(1, SEQ, VOCAB)


# speedup vs baseline: 1.1643x; 1.0159x over previous
import functools

import jax
import jax.numpy as jnp
from jax import lax
from jax.experimental import pallas as pl
from jax.experimental.pallas import tpu as pltpu
from jax.experimental.pallas import tpu_sc as plsc

VOCAB = 50257
DIM = 512
NUM_HEADS = 8
HEAD_DIM = 64
NUM_EXPERTS = 8
D_FF = 1024
SEQ = 2048

TOK_BLK = 256
Q_BLK = 512
V_BLK = 2048
GATHER_PER_STEP = 32


def _embed_gather_sc(table, ids):
    nw = 32
    b_per_w = SEQ // nw
    mesh = plsc.VectorSubcoreMesh(core_axis_name="c", subcore_axis_name="s")

    @functools.partial(
        pl.kernel, mesh=mesh,
        out_type=jax.ShapeDtypeStruct((SEQ, DIM), jnp.float32),
        compiler_params=pltpu.CompilerParams(use_tc_tiling_on_sc=True),
        scratch_types=[
            pltpu.VMEM((b_per_w,), jnp.int32),
            pltpu.VMEM((b_per_w, DIM), jnp.float32),
            pltpu.SemaphoreType.DMA,
        ],
    )
    def k(ids_hbm, table_hbm, out_hbm, idx_v, rows_v, sem):
        wid = lax.axis_index("s") * 2 + lax.axis_index("c")
        base = wid * b_per_w
        pltpu.sync_copy(ids_hbm.at[pl.ds(base, b_per_w)], idx_v)
        pltpu.async_copy(table_hbm.at[idx_v], rows_v, sem).wait()
        pltpu.sync_copy(rows_v, out_hbm.at[pl.ds(base, b_per_w)])

    return k(ids, table)


def _embed_tc_body(ids_ref, *refs):
    i = pl.program_id(0)
    out_ref = refs[GATHER_PER_STEP]
    rows = []
    iota8 = lax.broadcasted_iota(jnp.int32, (8, DIM), 0)
    for j in range(GATHER_PER_STEP):
        sub = lax.rem(ids_ref[i * GATHER_PER_STEP + j], 8)
        blk = refs[j][...]
        rows.append(jnp.sum(jnp.where(iota8 == sub, blk, 0.0),
                            axis=0, keepdims=True))
    out_ref[...] = jnp.concatenate(rows, axis=0)


def _embed_gather_tc(table, ids):
    def tbl_spec(j):
        return pl.BlockSpec(
            (8, DIM),
            lambda i, ids_ref, j=j: (ids_ref[i * GATHER_PER_STEP + j] // 8,
                                     0))

    grid_spec = pltpu.PrefetchScalarGridSpec(
        num_scalar_prefetch=1,
        grid=(SEQ // GATHER_PER_STEP,),
        in_specs=[tbl_spec(j) for j in range(GATHER_PER_STEP)],
        out_specs=pl.BlockSpec((GATHER_PER_STEP, DIM),
                               lambda i, ids_ref: (i, 0)),
    )
    return pl.pallas_call(
        _embed_tc_body,
        grid_spec=grid_spec,
        out_shape=jax.ShapeDtypeStruct((SEQ, DIM), jnp.float32),
    )(ids, *([table] * GATHER_PER_STEP))


def _pre_body(x_ref, pos_ref, ow1_ref, ob1_ref, ow2_ref, ob2_ref,
              dw1_ref, db1_ref, dw2_ref, db2_ref, out_ref):
    x = x_ref[...] + pos_ref[...]
    ov = jax.nn.gelu(
        jnp.dot(x, ow1_ref[...], preferred_element_type=jnp.float32)
        + ob1_ref[...])
    ov = jnp.dot(ov, ow2_ref[...], preferred_element_type=jnp.float32) \
        + ob2_ref[...]
    x = x + 0.1 * ov
    xd = x
    for _ in range(5):
        h = jax.nn.gelu(
            jnp.dot(xd, dw1_ref[...], preferred_element_type=jnp.float32)
            + db1_ref[...])
        h = jnp.dot(h, dw2_ref[...], preferred_element_type=jnp.float32) \
            + db2_ref[...]
        xd = xd - 0.1 * h
    out_ref[...] = x + 0.2 * xd


def _pre_mlps(x, pos, p):
    full = lambda r, c: pl.BlockSpec((r, c), lambda i: (0, 0))
    return pl.pallas_call(
        _pre_body,
        grid=(SEQ // TOK_BLK,),
        in_specs=[
            pl.BlockSpec((TOK_BLK, DIM), lambda i: (i, 0)),
            pl.BlockSpec((TOK_BLK, DIM), lambda i: (i, 0)),
            full(DIM, 4 * DIM), full(1, 4 * DIM),
            full(4 * DIM, DIM), full(1, DIM),
            full(DIM, 2 * DIM), full(1, 2 * DIM),
            full(2 * DIM, DIM), full(1, DIM),
        ],
        out_specs=pl.BlockSpec((TOK_BLK, DIM), lambda i: (i, 0)),
        out_shape=jax.ShapeDtypeStruct((SEQ, DIM), jnp.float32),
    )(x, pos, p['ov_w1'], p['ov_b1'][None, :], p['ov_w2'],
      p['ov_b2'][None, :], p['df_w1'], p['df_b1'][None, :], p['df_w2'],
      p['df_b2'][None, :])


def _matmul_bias_body(x_ref, w_ref, b_ref, out_ref):
    out_ref[...] = jnp.dot(x_ref[...], w_ref[...],
                           preferred_element_type=jnp.float32) + b_ref[...]


def _matmul_bias(x, w, b):
    n = w.shape[1]
    return pl.pallas_call(
        _matmul_bias_body,
        grid=(SEQ // TOK_BLK,),
        in_specs=[
            pl.BlockSpec((TOK_BLK, DIM), lambda i: (i, 0)),
            pl.BlockSpec((DIM, n), lambda i: (0, 0)),
            pl.BlockSpec((1, n), lambda i: (0, 0)),
        ],
        out_specs=pl.BlockSpec((TOK_BLK, n), lambda i: (i, 0)),
        out_shape=jax.ShapeDtypeStruct((SEQ, n), jnp.float32),
    )(x, w, b[None, :])


def _attn_body(q_ref, k_ref, v_ref, out_ref):
    qb = pl.program_id(0)
    q_all = q_ref[...]
    k_all = k_ref[...]
    v_all = v_ref[...]
    row = qb * Q_BLK + lax.broadcasted_iota(jnp.int32, (Q_BLK, SEQ), 0)
    col = lax.broadcasted_iota(jnp.int32, (Q_BLK, SEQ), 1)
    causal = row >= col
    outs = []
    for h in range(NUM_HEADS):
        q = q_all[:, h * HEAD_DIM:(h + 1) * HEAD_DIM]
        k = k_all[:, h * HEAD_DIM:(h + 1) * HEAD_DIM]
        v = v_all[:, h * HEAD_DIM:(h + 1) * HEAD_DIM]
        s = lax.dot_general(q, k, (((1,), (1,)), ((), ())),
                            preferred_element_type=jnp.float32) / 8.0
        s = jnp.where(causal, s, jnp.float32(-1e9))
        m = jnp.max(s, axis=-1, keepdims=True)
        e = jnp.exp(s - m)
        p = e / jnp.sum(e, axis=-1, keepdims=True)
        outs.append(jnp.dot(p, v, preferred_element_type=jnp.float32))
    out_ref[...] = jnp.concatenate(outs, axis=1)


def _attention(x, lp):
    qkv = _matmul_bias(x, lp['wqkv'], lp['bqkv'])
    heads = pl.pallas_call(
        _attn_body,
        grid=(SEQ // Q_BLK,),
        in_specs=[
            pl.BlockSpec((Q_BLK, DIM), lambda qb: (qb, 0)),
            pl.BlockSpec((SEQ, DIM), lambda qb: (0, 1)),
            pl.BlockSpec((SEQ, DIM), lambda qb: (0, 2)),
        ],
        out_specs=pl.BlockSpec((Q_BLK, DIM), lambda qb: (qb, 0)),
        out_shape=jax.ShapeDtypeStruct((SEQ, DIM), jnp.float32),
    )(qkv, qkv, qkv)
    return heads


def _proj_residual_body(h_ref, w_ref, b_ref, x_ref, out_ref):
    out_ref[...] = (jnp.dot(h_ref[...], w_ref[...],
                            preferred_element_type=jnp.float32)
                    + b_ref[...] + x_ref[...])


def _proj_residual(heads, w, b, x):
    return pl.pallas_call(
        _proj_residual_body,
        grid=(SEQ // TOK_BLK,),
        in_specs=[
            pl.BlockSpec((TOK_BLK, DIM), lambda i: (i, 0)),
            pl.BlockSpec((DIM, DIM), lambda i: (0, 0)),
            pl.BlockSpec((1, DIM), lambda i: (0, 0)),
            pl.BlockSpec((TOK_BLK, DIM), lambda i: (i, 0)),
        ],
        out_specs=pl.BlockSpec((TOK_BLK, DIM), lambda i: (i, 0)),
        out_shape=jax.ShapeDtypeStruct((SEQ, DIM), jnp.float32),
    )(heads, w, b[None, :], x)


def _moe_body(x_ref, wr_ref, w1_ref, b1_ref, w2_ref, b2_ref, out_ref):
    e = pl.program_id(0)
    x = x_ref[...]
    logits = jnp.dot(x, wr_ref[...], preferred_element_type=jnp.float32)
    lm = jnp.max(logits, axis=-1, keepdims=True)
    ex = jnp.exp(logits - lm)
    probs = ex / jnp.sum(ex, axis=-1, keepdims=True)
    iota = lax.broadcasted_iota(jnp.int32, probs.shape, 1)
    m1 = jnp.max(probs, axis=-1, keepdims=True)
    i1 = jnp.min(jnp.where(probs == m1, iota, NUM_EXPERTS),
                 axis=-1, keepdims=True)
    pm = jnp.where(iota == i1, jnp.float32(-1.0), probs)
    m2 = jnp.max(pm, axis=-1, keepdims=True)
    i2 = jnp.min(jnp.where(pm == m2, iota, NUM_EXPERTS),
                 axis=-1, keepdims=True)
    denom = m1 + m2
    w_e = jnp.where(i1 == e, m1 / denom,
                    jnp.where(i2 == e, m2 / denom, jnp.float32(0.0)))

    h = jax.nn.gelu(
        jnp.dot(x, w1_ref[0], preferred_element_type=jnp.float32)
        + b1_ref[0])
    y = jnp.dot(h, w2_ref[0], preferred_element_type=jnp.float32) \
        + b2_ref[0]

    @pl.when(e == 0)
    def _():
        out_ref[...] = x

    out_ref[...] += w_e * y


def _mini_moe(x, mp):
    return pl.pallas_call(
        _moe_body,
        grid=(NUM_EXPERTS,),
        in_specs=[
            pl.BlockSpec((SEQ, DIM), lambda e: (0, 0)),
            pl.BlockSpec((DIM, NUM_EXPERTS), lambda e: (0, 0)),
            pl.BlockSpec((1, DIM, D_FF), lambda e: (e, 0, 0)),
            pl.BlockSpec((1, 1, D_FF), lambda e: (e, 0, 0)),
            pl.BlockSpec((1, D_FF, DIM), lambda e: (e, 0, 0)),
            pl.BlockSpec((1, 1, DIM), lambda e: (e, 0, 0)),
        ],
        out_specs=pl.BlockSpec((SEQ, DIM), lambda e: (0, 0)),
        out_shape=jax.ShapeDtypeStruct((SEQ, DIM), jnp.float32),
    )(x, mp['wr'], mp['w1'], mp['b1'][:, None, :], mp['w2'],
      mp['b2'][:, None, :])


def _final_body(x_ref, g_ref, b_ref, tbl_ref, out_ref):
    x = x_ref[...]
    mu = jnp.mean(x, axis=-1, keepdims=True)
    var = jnp.mean((x - mu) ** 2, axis=-1, keepdims=True)
    xn = (x - mu) / jnp.sqrt(var + 1e-5) * g_ref[...] + b_ref[...]
    out_ref[...] = lax.dot_general(xn, tbl_ref[...],
                                   (((1,), (1,)), ((), ())),
                                   preferred_element_type=jnp.float32)


def _final_logits(x, g, b, table):
    return pl.pallas_call(
        _final_body,
        grid=(pl.cdiv(VOCAB, V_BLK),),
        in_specs=[
            pl.BlockSpec((SEQ, DIM), lambda j: (0, 0)),
            pl.BlockSpec((1, DIM), lambda j: (0, 0)),
            pl.BlockSpec((1, DIM), lambda j: (0, 0)),
            pl.BlockSpec((V_BLK, DIM), lambda j: (j, 0)),
        ],
        out_specs=pl.BlockSpec((SEQ, V_BLK), lambda j: (0, j)),
        out_shape=jax.ShapeDtypeStruct((SEQ, VOCAB), jnp.float32),
    )(x, g[None, :], b[None, :], table)


def kernel(params, input_ids):
    ids = input_ids.reshape(SEQ).astype(jnp.int32)
    pos = params['pos_embed'][0, :SEQ, :]
    x = _embed_gather_sc(params['token_embed'], ids)
    x = _pre_mlps(x, pos, params)
    for lp in params['layers']:
        heads = _attention(x, lp)
        x = _proj_residual(heads, lp['wo'], lp['bo'], x)
        for mp in lp['moes']:
            x = _mini_moe(x, mp)
    logits = _final_logits(x, params['ln_g'], params['ln_b'],
                           params['token_embed'])
    return logits.reshape(1, SEQ, VOCAB)
